# baseline (speedup 1.0000x reference)
import jax
import jax.numpy as jnp
from jax.experimental import pallas as pl
from jax.experimental.pallas import tpu as pltpu
from jax import lax


_N = 2097152
_NF = _N // 16
_BG = 2048
_NCH = _NF // _BG


def _body(tokbig_ref, wbig_ref, bbig_ref, x_any, o_any,
          xa, xb, ya, yb, fill,
          sxa, sxb, sya, syb, sf):
    xview = x_any
    oview = o_any

    wbig = wbig_ref[...]
    bbig = bbig_ref[...]
    row0 = jnp.dot(tokbig_ref[...], wbig,
                   preferred_element_type=jnp.float32) + bbig
    fill[...] = jnp.broadcast_to(row0, (_BG, 256))

    def xsrc(it):
        return xview.at[1, pl.ds(it * _BG, _BG), :]

    def ydst(it):
        return oview.at[1, pl.ds(it * _BG, _BG), :]

    def fdst(it):
        return oview.at[0, pl.ds(it * _BG, _BG), :]

    pltpu.make_async_copy(xsrc(0), xa, sxa).start()
    pltpu.make_async_copy(xsrc(1), xb, sxb).start()

    def pair_body(p, carry):
        it0 = 2 * p
        it1 = 2 * p + 1

        pltpu.make_async_copy(xsrc(it0), xa, sxa).wait()

        @pl.when(p >= 1)
        def _():
            pltpu.make_async_copy(ya, ydst(it0), sya).wait()
            pltpu.make_async_copy(fill, fdst(it0), sf).wait()

        ya[...] = jnp.dot(xa[...], wbig,
                          preferred_element_type=jnp.float32) + bbig
        pltpu.make_async_copy(ya, ydst(it0), sya).start()
        pltpu.make_async_copy(fill, fdst(it0), sf).start()

        @pl.when(p + 1 < _NCH // 2)
        def _():
            pltpu.make_async_copy(xsrc(it0 + 2), xa, sxa).start()

        pltpu.make_async_copy(xsrc(it1), xb, sxb).wait()

        @pl.when(p >= 1)
        def _():
            pltpu.make_async_copy(yb, ydst(it1), syb).wait()
            pltpu.make_async_copy(fill, fdst(it1), sf).wait()

        yb[...] = jnp.dot(xb[...], wbig,
                          preferred_element_type=jnp.float32) + bbig
        pltpu.make_async_copy(yb, ydst(it1), syb).start()
        pltpu.make_async_copy(fill, fdst(it1), sf).start()

        @pl.when(p + 1 < _NCH // 2)
        def _():
            pltpu.make_async_copy(xsrc(it1 + 2), xb, sxb).start()

        return carry

    lax.fori_loop(0, _NCH // 2, pair_body, 0)

    last0 = _NCH - 2
    last1 = _NCH - 1
    pltpu.make_async_copy(ya, ydst(last0), sya).wait()
    pltpu.make_async_copy(yb, ydst(last1), syb).wait()
    pltpu.make_async_copy(fill, fdst(last0), sf).wait()
    pltpu.make_async_copy(fill, fdst(last1), sf).wait()


def kernel(x, token, W, b):
    xv = x.reshape(2, _NF, 128)
    wt = W.T
    wbig = jnp.kron(jnp.eye(16, dtype=jnp.float32), wt)
    bbig = jnp.tile(b, 16).reshape(1, 256)
    tokbig = jnp.tile(token, 16).reshape(1, 128)
    out = pl.pallas_call(
        _body,
        in_specs=[
            pl.BlockSpec(memory_space=pltpu.MemorySpace.VMEM),
            pl.BlockSpec(memory_space=pltpu.MemorySpace.VMEM),
            pl.BlockSpec(memory_space=pltpu.MemorySpace.VMEM),
            pl.BlockSpec(memory_space=pltpu.MemorySpace.HBM),
        ],
        out_specs=pl.BlockSpec(memory_space=pltpu.MemorySpace.HBM),
        out_shape=jax.ShapeDtypeStruct((2, _NF, 256), jnp.float32),
        scratch_shapes=[
            pltpu.VMEM((_BG, 128), jnp.float32),
            pltpu.VMEM((_BG, 128), jnp.float32),
            pltpu.VMEM((_BG, 256), jnp.float32),
            pltpu.VMEM((_BG, 256), jnp.float32),
            pltpu.VMEM((_BG, 256), jnp.float32),
            pltpu.SemaphoreType.DMA,
            pltpu.SemaphoreType.DMA,
            pltpu.SemaphoreType.DMA,
            pltpu.SemaphoreType.DMA,
            pltpu.SemaphoreType.DMA,
        ],
    )(tokbig, wbig, bbig, xv)
    return out.reshape(2, _N, 16)
